# Initial kernel scaffold; baseline (speedup 1.0000x reference)
import functools

import jax
import jax.numpy as jnp
from jax import lax
from jax.experimental import pallas as pl
from jax.experimental.pallas import tpu as pltpu
from jax.experimental.pallas import tpu_sc as plsc

N = 10000
E = 320000
D_IN = 128
H1 = 128
H2 = 64
K = 16

NC = 2
NS = 16
NW = NC * NS
LANES = 16

CHUNK = 128
EPT_RAW = E // NW
NCH = 80
EPT = NCH * CHUNK
NPAD = 10240
RPT = NPAD // NS
TRASH = N

BR = 1000

_MESH = plsc.VectorSubcoreMesh(core_axis_name="c", subcore_axis_name="s")


def _sc_aggregate(d):
    @functools.partial(
        pl.kernel,
        mesh=_MESH,
        out_type=jax.ShapeDtypeStruct((NC, NPAD, d), jnp.float32),
        scratch_types=[
            pltpu.VMEM((NCH, CHUNK), jnp.int32),
            pltpu.VMEM((NCH, CHUNK), jnp.int32),
            pltpu.VMEM((CHUNK, d), jnp.float32),
            pltpu.VMEM_SHARED((NPAD, d), jnp.float32),
            pltpu.SemaphoreType.DMA,
        ],
    )
    def k(hp_hbm, src_hbm, dst_hbm, zeros_hbm, out_hbm,
          src_v, dst_v, rows_v, acc_sh, sem):
        cid = lax.axis_index("c")
        sid = lax.axis_index("s")
        wid = sid * NC + cid

        pltpu.sync_copy(zeros_hbm, rows_v)

        @pl.loop(0, RPT, step=CHUNK)
        def _(r):
            pltpu.sync_copy(rows_v, acc_sh.at[pl.ds(sid * RPT + r, CHUNK)])

        pltpu.sync_copy(src_hbm.at[wid], src_v)
        pltpu.sync_copy(dst_hbm.at[wid], dst_v)
        plsc.subcore_barrier()

        @pl.loop(0, NCH)
        def _(j):
            pltpu.async_copy(hp_hbm.at[src_v.at[j]], rows_v, sem).wait()
            pltpu.sync_copy(rows_v, acc_sh.at[dst_v.at[j]], add=True)

        plsc.subcore_barrier()

        @pl.loop(0, RPT, step=CHUNK)
        def _(r):
            pltpu.sync_copy(acc_sh.at[pl.ds(sid * RPT + r, CHUNK)],
                            out_hbm.at[cid].at[pl.ds(sid * RPT + r, CHUNK)])

    return k


def _sc_degree():
    @functools.partial(
        pl.kernel,
        mesh=_MESH,
        out_type=jax.ShapeDtypeStruct((NC, NPAD, LANES), jnp.float32),
        scratch_types=[
            pltpu.VMEM((NCH, CHUNK), jnp.int32),
            pltpu.VMEM((CHUNK, LANES), jnp.float32),
            pltpu.VMEM((CHUNK, LANES), jnp.float32),
            pltpu.VMEM_SHARED((NPAD, LANES), jnp.float32),
        ],
    )
    def k(dst_hbm, zeros_hbm, ones_hbm, out_hbm, dst_v, zrows_v, ones_v, acc_sh):
        cid = lax.axis_index("c")
        sid = lax.axis_index("s")
        wid = sid * NC + cid

        pltpu.sync_copy(zeros_hbm, zrows_v)
        pltpu.sync_copy(ones_hbm, ones_v)

        @pl.loop(0, RPT, step=CHUNK)
        def _(r):
            pltpu.sync_copy(zrows_v, acc_sh.at[pl.ds(sid * RPT + r, CHUNK)])

        pltpu.sync_copy(dst_hbm.at[wid], dst_v)
        plsc.subcore_barrier()

        @pl.loop(0, NCH)
        def _(j):
            pltpu.sync_copy(ones_v, acc_sh.at[dst_v.at[j]], add=True)

        plsc.subcore_barrier()

        @pl.loop(0, RPT, step=CHUNK)
        def _(r):
            pltpu.sync_copy(acc_sh.at[pl.ds(sid * RPT + r, CHUNK)],
                            out_hbm.at[cid].at[pl.ds(sid * RPT + r, CHUNK)])

    return k


def _dot(a, b):
    return jnp.dot(a, b, precision=lax.Precision.HIGHEST,
                   preferred_element_type=jnp.float32)


def _tc_stage1(x, w1, dp):
    def body(x_ref, w_ref, dp_ref, hp_ref, dinv_ref):
        dp = dp_ref[...]
        deg = dp[0, :, 0:1] + dp[1, :, 0:1] + 1.0
        dinv = lax.rsqrt(deg)
        g = _dot(x_ref[...], w_ref[...])
        hp_ref[...] = dinv * g
        dinv_ref[...] = dinv

    return pl.pallas_call(
        body,
        grid=(N // BR,),
        in_specs=[
            pl.BlockSpec((BR, D_IN), lambda i: (i, 0)),
            pl.BlockSpec((D_IN, H1), lambda i: (0, 0)),
            pl.BlockSpec((NC, BR, LANES), lambda i: (0, i, 0)),
        ],
        out_specs=[
            pl.BlockSpec((BR, H1), lambda i: (i, 0)),
            pl.BlockSpec((BR, 1), lambda i: (i, 0)),
        ],
        out_shape=[
            jax.ShapeDtypeStruct((N, H1), jnp.float32),
            jax.ShapeDtypeStruct((N, 1), jnp.float32),
        ],
    )(x, w1, dp)


def _tc_stage_mid(acc, hp, dinv, b, wn, d, dn):
    def body(acc_ref, hp_ref, dinv_ref, b_ref, w_ref, out_ref):
        a = acc_ref[...]
        dinv = dinv_ref[...]
        s = a[0] + a[1] + hp_ref[...]
        h = jnp.maximum(dinv * s + b_ref[...], 0.0)
        out_ref[...] = dinv * _dot(h, w_ref[...])

    return pl.pallas_call(
        body,
        grid=(N // BR,),
        in_specs=[
            pl.BlockSpec((NC, BR, d), lambda i: (0, i, 0)),
            pl.BlockSpec((BR, d), lambda i: (i, 0)),
            pl.BlockSpec((BR, 1), lambda i: (i, 0)),
            pl.BlockSpec((1, d), lambda i: (0, 0)),
            pl.BlockSpec((d, dn), lambda i: (0, 0)),
        ],
        out_specs=pl.BlockSpec((BR, dn), lambda i: (i, 0)),
        out_shape=jax.ShapeDtypeStruct((N, dn), jnp.float32),
    )(acc, hp, dinv, b, wn)


def _tc_stage3(acc, hp, dinv, b):
    def body(acc_ref, hp_ref, dinv_ref, b_ref, out_ref):
        a = acc_ref[...]
        logits = dinv_ref[...] * (a[0] + a[1] + hp_ref[...]) + b_ref[...]
        m = jnp.max(logits, axis=1, keepdims=True)
        e = jnp.exp(logits - m)
        out_ref[...] = e / jnp.sum(e, axis=1, keepdims=True)

    return pl.pallas_call(
        body,
        grid=(N // BR,),
        in_specs=[
            pl.BlockSpec((NC, BR, K), lambda i: (0, i, 0)),
            pl.BlockSpec((BR, K), lambda i: (i, 0)),
            pl.BlockSpec((BR, 1), lambda i: (i, 0)),
            pl.BlockSpec((1, K), lambda i: (0, 0)),
        ],
        out_specs=pl.BlockSpec((BR, K), lambda i: (i, 0)),
        out_shape=jax.ShapeDtypeStruct((N, K), jnp.float32),
    )(acc, hp, dinv, b)


def kernel(x, edge_index, W1, b1, W2, b2, W3, b3):
    src = edge_index[0].reshape(NW, EPT_RAW)
    dst = edge_index[1].reshape(NW, EPT_RAW)
    pad = EPT - EPT_RAW
    src = jnp.pad(src, ((0, 0), (0, pad))).reshape(NW, NCH, CHUNK)
    dst = jnp.pad(dst, ((0, 0), (0, pad)),
                  constant_values=TRASH).reshape(NW, NCH, CHUNK)

    z16 = jnp.zeros((CHUNK, LANES), jnp.float32)
    ones16 = jnp.ones((CHUNK, LANES), jnp.float32)
    z128 = jnp.zeros((CHUNK, H1), jnp.float32)
    z64 = jnp.zeros((CHUNK, H2), jnp.float32)

    dp = _sc_degree()(dst, z16, ones16)

    hp1, dinv = _tc_stage1(x, W1, dp)
    acc1 = _sc_aggregate(H1)(hp1, src, dst, z128)
    hp2 = _tc_stage_mid(acc1, hp1, dinv, b1.reshape(1, H1), W2, H1, H2)
    acc2 = _sc_aggregate(H2)(hp2, src, dst, z64)
    hp3 = _tc_stage_mid(acc2, hp2, dinv, b2.reshape(1, H2), W3, H2, K)
    acc3 = _sc_aggregate(K)(hp3, src, dst, z16)

    return _tc_stage3(acc3, hp3, dinv, b3.reshape(1, K))


# speedup vs baseline: 7.7525x; 7.7525x over previous
import functools

import jax
import jax.numpy as jnp
from jax import lax
from jax.experimental import pallas as pl
from jax.experimental.pallas import tpu as pltpu
from jax.experimental.pallas import tpu_sc as plsc

N = 10000
E = 320000
D_IN = 128
H1 = 128
H2 = 64
K = 16

NC = 2
NS = 16
NW = NC * NS
LANES = 16

CHUNK = 128
EPT_RAW = E // NW
NCH = 80
EPT = NCH * CHUNK
NPAD = 10240
RPT = NPAD // NS
TRASH = N

DW = 128
BR = 1000

_MESH = plsc.VectorSubcoreMesh(core_axis_name="c", subcore_axis_name="s")


def _sc_aggregate():
    d = DW

    @functools.partial(
        pl.kernel,
        mesh=_MESH,
        out_type=jax.ShapeDtypeStruct((NC, NPAD, d), jnp.float32),
        scratch_types=[
            pltpu.VMEM((NCH, CHUNK), jnp.int32),
            pltpu.VMEM((NCH, CHUNK), jnp.int32),
            pltpu.VMEM((CHUNK, d), jnp.float32),
            pltpu.VMEM_SHARED((NPAD, d), jnp.float32),
            pltpu.SemaphoreType.DMA,
        ],
    )
    def k(hp_hbm, src_hbm, dst_hbm, zeros_hbm, out_hbm,
          src_v, dst_v, rows_v, acc_sh, sem):
        cid = lax.axis_index("c")
        sid = lax.axis_index("s")
        wid = sid * NC + cid

        pltpu.sync_copy(zeros_hbm, rows_v)

        @pl.loop(0, RPT, step=CHUNK)
        def _(r):
            pltpu.sync_copy(rows_v, acc_sh.at[pl.ds(sid * RPT + r, CHUNK)])

        pltpu.sync_copy(src_hbm.at[wid], src_v)
        pltpu.sync_copy(dst_hbm.at[wid], dst_v)
        plsc.subcore_barrier()

        @pl.loop(0, NCH)
        def _(j):
            pltpu.async_copy(hp_hbm.at[src_v.at[j]], rows_v, sem).wait()
            pltpu.sync_copy(rows_v, acc_sh.at[dst_v.at[j]], add=True)

        plsc.subcore_barrier()

        @pl.loop(0, RPT, step=CHUNK)
        def _(r):
            pltpu.sync_copy(acc_sh.at[pl.ds(sid * RPT + r, CHUNK)],
                            out_hbm.at[cid].at[pl.ds(sid * RPT + r, CHUNK)])

    return k


def _sc_degree():
    @functools.partial(
        pl.kernel,
        mesh=_MESH,
        out_type=jax.ShapeDtypeStruct((NC, NPAD, DW), jnp.float32),
        scratch_types=[
            pltpu.VMEM((NCH, CHUNK), jnp.int32),
            pltpu.VMEM((CHUNK, DW), jnp.float32),
            pltpu.VMEM((CHUNK, DW), jnp.float32),
            pltpu.VMEM_SHARED((NPAD, DW), jnp.float32),
        ],
    )
    def k(dst_hbm, zeros_hbm, ones_hbm, out_hbm, dst_v, zrows_v, ones_v, acc_sh):
        cid = lax.axis_index("c")
        sid = lax.axis_index("s")
        wid = sid * NC + cid

        pltpu.sync_copy(zeros_hbm, zrows_v)
        pltpu.sync_copy(ones_hbm, ones_v)

        @pl.loop(0, RPT, step=CHUNK)
        def _(r):
            pltpu.sync_copy(zrows_v, acc_sh.at[pl.ds(sid * RPT + r, CHUNK)])

        pltpu.sync_copy(dst_hbm.at[wid], dst_v)
        plsc.subcore_barrier()

        @pl.loop(0, NCH)
        def _(j):
            pltpu.sync_copy(ones_v, acc_sh.at[dst_v.at[j]], add=True)

        plsc.subcore_barrier()

        @pl.loop(0, RPT, step=CHUNK)
        def _(r):
            pltpu.sync_copy(acc_sh.at[pl.ds(sid * RPT + r, CHUNK)],
                            out_hbm.at[cid].at[pl.ds(sid * RPT + r, CHUNK)])

    return k


def _dot(a, b):
    return jnp.dot(a, b, precision=lax.Precision.HIGHEST,
                   preferred_element_type=jnp.float32)


def _tc_stage1(x, w1, dp):
    def body(x_ref, w_ref, dp_ref, hp_ref, dinv_ref):
        dp = dp_ref[...]
        deg = dp[0, :, 0:1] + dp[1, :, 0:1] + 1.0
        dinv = lax.rsqrt(deg)
        g = _dot(x_ref[...], w_ref[...])
        hp_ref[...] = dinv * g
        dinv_ref[...] = dinv

    return pl.pallas_call(
        body,
        grid=(N // BR,),
        in_specs=[
            pl.BlockSpec((BR, D_IN), lambda i: (i, 0)),
            pl.BlockSpec((D_IN, H1), lambda i: (0, 0)),
            pl.BlockSpec((NC, BR, DW), lambda i: (0, i, 0)),
        ],
        out_specs=[
            pl.BlockSpec((BR, H1), lambda i: (i, 0)),
            pl.BlockSpec((BR, 1), lambda i: (i, 0)),
        ],
        out_shape=[
            jax.ShapeDtypeStruct((N, H1), jnp.float32),
            jax.ShapeDtypeStruct((N, 1), jnp.float32),
        ],
    )(x, w1, dp)


def _tc_stage_mid(acc, hp, dinv, b, wn):
    def body(acc_ref, hp_ref, dinv_ref, b_ref, w_ref, out_ref):
        a = acc_ref[...]
        dinv = dinv_ref[...]
        s = a[0] + a[1] + hp_ref[...]
        h = jnp.maximum(dinv * s + b_ref[...], 0.0)
        out_ref[...] = dinv * _dot(h, w_ref[...])

    return pl.pallas_call(
        body,
        grid=(N // BR,),
        in_specs=[
            pl.BlockSpec((NC, BR, DW), lambda i: (0, i, 0)),
            pl.BlockSpec((BR, DW), lambda i: (i, 0)),
            pl.BlockSpec((BR, 1), lambda i: (i, 0)),
            pl.BlockSpec((1, DW), lambda i: (0, 0)),
            pl.BlockSpec((DW, DW), lambda i: (0, 0)),
        ],
        out_specs=pl.BlockSpec((BR, DW), lambda i: (i, 0)),
        out_shape=jax.ShapeDtypeStruct((N, DW), jnp.float32),
    )(acc, hp, dinv, b, wn)


def _tc_stage3(acc, hp, dinv, b):
    def body(acc_ref, hp_ref, dinv_ref, b_ref, out_ref):
        a = acc_ref[...]
        full = dinv_ref[...] * (a[0] + a[1] + hp_ref[...])
        logits = full[:, 0:K] + b_ref[...]
        m = jnp.max(logits, axis=1, keepdims=True)
        e = jnp.exp(logits - m)
        out_ref[...] = e / jnp.sum(e, axis=1, keepdims=True)

    return pl.pallas_call(
        body,
        grid=(N // BR,),
        in_specs=[
            pl.BlockSpec((NC, BR, DW), lambda i: (0, i, 0)),
            pl.BlockSpec((BR, DW), lambda i: (i, 0)),
            pl.BlockSpec((BR, 1), lambda i: (i, 0)),
            pl.BlockSpec((1, K), lambda i: (0, 0)),
        ],
        out_specs=pl.BlockSpec((BR, K), lambda i: (i, 0)),
        out_shape=jax.ShapeDtypeStruct((N, K), jnp.float32),
    )(acc, hp, dinv, b)


def kernel(x, edge_index, W1, b1, W2, b2, W3, b3):
    src = edge_index[0].reshape(NW, EPT_RAW)
    dst = edge_index[1].reshape(NW, EPT_RAW)
    pad = EPT - EPT_RAW
    src = jnp.pad(src, ((0, 0), (0, pad))).reshape(NW, NCH, CHUNK)
    dst = jnp.pad(dst, ((0, 0), (0, pad)),
                  constant_values=TRASH).reshape(NW, NCH, CHUNK)

    zdw = jnp.zeros((CHUNK, DW), jnp.float32)
    onesdw = jnp.ones((CHUNK, DW), jnp.float32)

    w2p = jnp.zeros((DW, DW), jnp.float32).at[:H1, :H2].set(W2)
    w3p = jnp.zeros((DW, DW), jnp.float32).at[:H2, :K].set(W3)
    b2p = jnp.zeros((1, DW), jnp.float32).at[0, :H2].set(b2)

    agg = _sc_aggregate()

    dp = _sc_degree()(dst, zdw, onesdw)

    hp1, dinv = _tc_stage1(x, W1, dp)
    acc1 = agg(hp1, src, dst, zdw)
    hp2 = _tc_stage_mid(acc1, hp1, dinv, b1.reshape(1, H1), w2p)
    acc2 = agg(hp2, src, dst, zdw)
    hp3 = _tc_stage_mid(acc2, hp2, dinv, b2p, w3p)
    acc3 = agg(hp3, src, dst, zdw)

    return _tc_stage3(acc3, hp3, dinv, b3.reshape(1, K))


# speedup vs baseline: 8.7977x; 1.1348x over previous
import functools

import jax
import jax.numpy as jnp
from jax import lax
from jax.experimental import pallas as pl
from jax.experimental.pallas import tpu as pltpu
from jax.experimental.pallas import tpu_sc as plsc

N = 10000
E = 320000
D_IN = 128
H1 = 128
H2 = 64
K = 16

NC = 2
NS = 16
NW = NC * NS
LANES = 16

CHUNK = 128
EPT_RAW = E // NW
NCH = 80
EPT = NCH * CHUNK
NPAD = 10240
RPT = NPAD // NS
TRASH = N

DW = 128
BR = 1000

_MESH = plsc.VectorSubcoreMesh(core_axis_name="c", subcore_axis_name="s")


NBUF = 2
NSEG = 2
NCHS = NCH // NSEG


def _sc_aggregate():
    d = DW

    @functools.partial(
        pl.kernel,
        mesh=_MESH,
        out_type=jax.ShapeDtypeStruct((NC, NPAD, d), jnp.float32),
        scratch_types=[
            pltpu.VMEM((NCHS, CHUNK), jnp.int32),
            pltpu.VMEM((NCHS, CHUNK), jnp.int32),
            pltpu.VMEM((NBUF, CHUNK, d), jnp.float32),
            pltpu.VMEM_SHARED((NPAD, d), jnp.float32),
        ] + [pltpu.SemaphoreType.DMA] * NBUF,
    )
    def k(hp_hbm, src_hbm, dst_hbm, zeros_hbm, out_hbm,
          src_v, dst_v, rows_v, acc_sh, *sems):
        cid = lax.axis_index("c")
        sid = lax.axis_index("s")
        wid = sid * NC + cid

        pltpu.sync_copy(zeros_hbm, rows_v.at[0])

        @pl.loop(0, RPT, step=CHUNK)
        def _(r):
            pltpu.sync_copy(rows_v.at[0], acc_sh.at[pl.ds(sid * RPT + r, CHUNK)])

        plsc.subcore_barrier()

        @pl.loop(0, NSEG)
        def _(seg):
            pltpu.sync_copy(src_hbm.at[wid].at[seg], src_v)
            pltpu.sync_copy(dst_hbm.at[wid].at[seg], dst_v)

            for b in range(NBUF):
                pltpu.async_copy(hp_hbm.at[src_v.at[b]], rows_v.at[b], sems[b])

            @pl.loop(0, NCHS, step=NBUF)
            def _(j):
                for b in range(NBUF):
                    pltpu.make_async_copy(zeros_hbm, rows_v.at[b],
                                          sems[b]).wait()
                    pltpu.sync_copy(rows_v.at[b], acc_sh.at[dst_v.at[j + b]],
                                    add=True)

                    @pl.when(j + b + NBUF < NCHS)
                    def _():
                        pltpu.async_copy(hp_hbm.at[src_v.at[j + b + NBUF]],
                                         rows_v.at[b], sems[b])

        plsc.subcore_barrier()

        @pl.loop(0, RPT, step=CHUNK)
        def _(r):
            pltpu.sync_copy(acc_sh.at[pl.ds(sid * RPT + r, CHUNK)],
                            out_hbm.at[cid].at[pl.ds(sid * RPT + r, CHUNK)])

    return k


def _sc_degree():
    @functools.partial(
        pl.kernel,
        mesh=_MESH,
        out_type=jax.ShapeDtypeStruct((NC, NPAD, DW), jnp.float32),
        scratch_types=[
            pltpu.VMEM((NCH, CHUNK), jnp.int32),
            pltpu.VMEM((CHUNK, DW), jnp.float32),
            pltpu.VMEM((CHUNK, DW), jnp.float32),
            pltpu.VMEM_SHARED((NPAD, DW), jnp.float32),
        ],
    )
    def k(dst_hbm, zeros_hbm, ones_hbm, out_hbm, dst_v, zrows_v, ones_v, acc_sh):
        cid = lax.axis_index("c")
        sid = lax.axis_index("s")
        wid = sid * NC + cid

        pltpu.sync_copy(zeros_hbm, zrows_v)
        pltpu.sync_copy(ones_hbm, ones_v)

        @pl.loop(0, RPT, step=CHUNK)
        def _(r):
            pltpu.sync_copy(zrows_v, acc_sh.at[pl.ds(sid * RPT + r, CHUNK)])

        pltpu.sync_copy(dst_hbm.at[wid], dst_v)
        plsc.subcore_barrier()

        @pl.loop(0, NCH)
        def _(j):
            pltpu.sync_copy(ones_v, acc_sh.at[dst_v.at[j]], add=True)

        plsc.subcore_barrier()

        @pl.loop(0, RPT, step=CHUNK)
        def _(r):
            pltpu.sync_copy(acc_sh.at[pl.ds(sid * RPT + r, CHUNK)],
                            out_hbm.at[cid].at[pl.ds(sid * RPT + r, CHUNK)])

    return k


def _dot(a, b):
    return jnp.dot(a, b, precision=lax.Precision.HIGHEST,
                   preferred_element_type=jnp.float32)


def _tc_stage1(x, w1, dp):
    def body(x_ref, w_ref, dp_ref, hp_ref, dinv_ref):
        dp = dp_ref[...]
        deg = dp[0, :, 0:1] + dp[1, :, 0:1] + 1.0
        dinv = lax.rsqrt(deg)
        g = _dot(x_ref[...], w_ref[...])
        hp_ref[...] = dinv * g
        dinv_ref[...] = dinv

    return pl.pallas_call(
        body,
        grid=(N // BR,),
        in_specs=[
            pl.BlockSpec((BR, D_IN), lambda i: (i, 0)),
            pl.BlockSpec((D_IN, H1), lambda i: (0, 0)),
            pl.BlockSpec((NC, BR, DW), lambda i: (0, i, 0)),
        ],
        out_specs=[
            pl.BlockSpec((BR, H1), lambda i: (i, 0)),
            pl.BlockSpec((BR, 1), lambda i: (i, 0)),
        ],
        out_shape=[
            jax.ShapeDtypeStruct((N, H1), jnp.float32),
            jax.ShapeDtypeStruct((N, 1), jnp.float32),
        ],
    )(x, w1, dp)


def _tc_stage_mid(acc, hp, dinv, b, wn):
    def body(acc_ref, hp_ref, dinv_ref, b_ref, w_ref, out_ref):
        a = acc_ref[...]
        dinv = dinv_ref[...]
        s = a[0] + a[1] + hp_ref[...]
        h = jnp.maximum(dinv * s + b_ref[...], 0.0)
        out_ref[...] = dinv * _dot(h, w_ref[...])

    return pl.pallas_call(
        body,
        grid=(N // BR,),
        in_specs=[
            pl.BlockSpec((NC, BR, DW), lambda i: (0, i, 0)),
            pl.BlockSpec((BR, DW), lambda i: (i, 0)),
            pl.BlockSpec((BR, 1), lambda i: (i, 0)),
            pl.BlockSpec((1, DW), lambda i: (0, 0)),
            pl.BlockSpec((DW, DW), lambda i: (0, 0)),
        ],
        out_specs=pl.BlockSpec((BR, DW), lambda i: (i, 0)),
        out_shape=jax.ShapeDtypeStruct((N, DW), jnp.float32),
    )(acc, hp, dinv, b, wn)


def _tc_stage3(acc, hp, dinv, b):
    def body(acc_ref, hp_ref, dinv_ref, b_ref, out_ref):
        a = acc_ref[...]
        full = dinv_ref[...] * (a[0] + a[1] + hp_ref[...])
        logits = full[:, 0:K] + b_ref[...]
        m = jnp.max(logits, axis=1, keepdims=True)
        e = jnp.exp(logits - m)
        out_ref[...] = e / jnp.sum(e, axis=1, keepdims=True)

    return pl.pallas_call(
        body,
        grid=(N // BR,),
        in_specs=[
            pl.BlockSpec((NC, BR, DW), lambda i: (0, i, 0)),
            pl.BlockSpec((BR, DW), lambda i: (i, 0)),
            pl.BlockSpec((BR, 1), lambda i: (i, 0)),
            pl.BlockSpec((1, K), lambda i: (0, 0)),
        ],
        out_specs=pl.BlockSpec((BR, K), lambda i: (i, 0)),
        out_shape=jax.ShapeDtypeStruct((N, K), jnp.float32),
    )(acc, hp, dinv, b)


def kernel(x, edge_index, W1, b1, W2, b2, W3, b3):
    src = edge_index[0].reshape(NW, EPT_RAW)
    dst = edge_index[1].reshape(NW, EPT_RAW)
    pad = EPT - EPT_RAW
    src = jnp.pad(src, ((0, 0), (0, pad))).reshape(NW, NSEG, NCHS, CHUNK)
    dst = jnp.pad(dst, ((0, 0), (0, pad)),
                  constant_values=TRASH).reshape(NW, NSEG, NCHS, CHUNK)

    zdw = jnp.zeros((CHUNK, DW), jnp.float32)
    onesdw = jnp.ones((CHUNK, DW), jnp.float32)

    w2p = jnp.zeros((DW, DW), jnp.float32).at[:H1, :H2].set(W2)
    w3p = jnp.zeros((DW, DW), jnp.float32).at[:H2, :K].set(W3)
    b2p = jnp.zeros((1, DW), jnp.float32).at[0, :H2].set(b2)

    agg = _sc_aggregate()

    dp = _sc_degree()(dst.reshape(NW, NCH, CHUNK), zdw, onesdw)

    hp1, dinv = _tc_stage1(x, W1, dp)
    acc1 = agg(hp1, src, dst, zdw)
    hp2 = _tc_stage_mid(acc1, hp1, dinv, b1.reshape(1, H1), w2p)
    acc2 = agg(hp2, src, dst, zdw)
    hp3 = _tc_stage_mid(acc2, hp2, dinv, b2p, w3p)
    acc3 = agg(hp3, src, dst, zdw)

    return _tc_stage3(acc3, hp3, dinv, b3.reshape(1, K))


# speedup vs baseline: 8.8041x; 1.0007x over previous
import functools

import jax
import jax.numpy as jnp
from jax import lax
from jax.experimental import pallas as pl
from jax.experimental.pallas import tpu as pltpu
from jax.experimental.pallas import tpu_sc as plsc

N = 10000
E = 320000
D_IN = 128
H1 = 128
H2 = 64
K = 16

NC = 2
NS = 16
NW = NC * NS
LANES = 16

CHUNK = 128
EPT_RAW = E // NW
NCH = 80
EPT = NCH * CHUNK
NPAD = 10240
RPT = NPAD // NS
TRASH = N

DW = 128
BR = 1000

_MESH = plsc.VectorSubcoreMesh(core_axis_name="c", subcore_axis_name="s")


NBUF = 2
NSEG = 2
NCHS = NCH // NSEG
HALF = CHUNK // 2


def _sc_aggregate():
    d = DW

    @functools.partial(
        pl.kernel,
        mesh=_MESH,
        out_type=jax.ShapeDtypeStruct((NC, NPAD, d), jnp.float32),
        scratch_types=[
            pltpu.VMEM((NCHS, CHUNK), jnp.int32),
            pltpu.VMEM((NCHS, CHUNK), jnp.int32),
            pltpu.VMEM((NBUF, CHUNK, d), jnp.float32),
            pltpu.VMEM_SHARED((NPAD, d), jnp.float32),
        ] + [pltpu.SemaphoreType.DMA] * NBUF,
    )
    def k(hp_hbm, src_hbm, dst_hbm, zeros_hbm, out_hbm,
          src_v, dst_v, rows_v, acc_sh, *sems):
        cid = lax.axis_index("c")
        sid = lax.axis_index("s")
        wid = sid * NC + cid

        pltpu.sync_copy(zeros_hbm, rows_v.at[0])

        @pl.loop(0, RPT, step=CHUNK)
        def _(r):
            pltpu.sync_copy(rows_v.at[0], acc_sh.at[pl.ds(sid * RPT + r, CHUNK)])

        plsc.subcore_barrier()

        @pl.loop(0, NSEG)
        def _(seg):
            pltpu.sync_copy(src_hbm.at[wid].at[seg], src_v)
            pltpu.sync_copy(dst_hbm.at[wid].at[seg], dst_v)

            def issue(c, b):
                for h in (0, HALF):
                    pltpu.async_copy(
                        hp_hbm.at[src_v.at[c].at[pl.ds(h, HALF)]],
                        rows_v.at[b].at[pl.ds(h, HALF)], sems[b])

            for b in range(NBUF):
                issue(b, b)

            @pl.loop(0, NCHS, step=NBUF)
            def _(j):
                for b in range(NBUF):
                    pltpu.make_async_copy(zeros_hbm, rows_v.at[b],
                                          sems[b]).wait()
                    pltpu.sync_copy(rows_v.at[b], acc_sh.at[dst_v.at[j + b]],
                                    add=True)

                    @pl.when(j + b + NBUF < NCHS)
                    def _():
                        issue(j + b + NBUF, b)

        plsc.subcore_barrier()

        @pl.loop(0, RPT, step=CHUNK)
        def _(r):
            pltpu.sync_copy(acc_sh.at[pl.ds(sid * RPT + r, CHUNK)],
                            out_hbm.at[cid].at[pl.ds(sid * RPT + r, CHUNK)])

    return k


def _sc_degree():
    @functools.partial(
        pl.kernel,
        mesh=_MESH,
        out_type=jax.ShapeDtypeStruct((NC, NPAD, DW), jnp.float32),
        scratch_types=[
            pltpu.VMEM((NCH, CHUNK), jnp.int32),
            pltpu.VMEM((CHUNK, DW), jnp.float32),
            pltpu.VMEM((CHUNK, DW), jnp.float32),
            pltpu.VMEM_SHARED((NPAD, DW), jnp.float32),
        ],
    )
    def k(dst_hbm, zeros_hbm, ones_hbm, out_hbm, dst_v, zrows_v, ones_v, acc_sh):
        cid = lax.axis_index("c")
        sid = lax.axis_index("s")
        wid = sid * NC + cid

        pltpu.sync_copy(zeros_hbm, zrows_v)
        pltpu.sync_copy(ones_hbm, ones_v)

        @pl.loop(0, RPT, step=CHUNK)
        def _(r):
            pltpu.sync_copy(zrows_v, acc_sh.at[pl.ds(sid * RPT + r, CHUNK)])

        pltpu.sync_copy(dst_hbm.at[wid], dst_v)
        plsc.subcore_barrier()

        @pl.loop(0, NCH)
        def _(j):
            pltpu.sync_copy(ones_v, acc_sh.at[dst_v.at[j]], add=True)

        plsc.subcore_barrier()

        @pl.loop(0, RPT, step=CHUNK)
        def _(r):
            pltpu.sync_copy(acc_sh.at[pl.ds(sid * RPT + r, CHUNK)],
                            out_hbm.at[cid].at[pl.ds(sid * RPT + r, CHUNK)])

    return k


def _dot(a, b):
    return jnp.dot(a, b, precision=lax.Precision.HIGHEST,
                   preferred_element_type=jnp.float32)


def _tc_stage1(x, w1, dp):
    def body(x_ref, w_ref, dp_ref, hp_ref, dinv_ref):
        dp = dp_ref[...]
        deg = dp[0, :, 0:1] + dp[1, :, 0:1] + 1.0
        dinv = lax.rsqrt(deg)
        g = _dot(x_ref[...], w_ref[...])
        hp_ref[...] = dinv * g
        dinv_ref[...] = dinv

    return pl.pallas_call(
        body,
        grid=(N // BR,),
        in_specs=[
            pl.BlockSpec((BR, D_IN), lambda i: (i, 0)),
            pl.BlockSpec((D_IN, H1), lambda i: (0, 0)),
            pl.BlockSpec((NC, BR, DW), lambda i: (0, i, 0)),
        ],
        out_specs=[
            pl.BlockSpec((BR, H1), lambda i: (i, 0)),
            pl.BlockSpec((BR, 1), lambda i: (i, 0)),
        ],
        out_shape=[
            jax.ShapeDtypeStruct((N, H1), jnp.float32),
            jax.ShapeDtypeStruct((N, 1), jnp.float32),
        ],
    )(x, w1, dp)


def _tc_stage_mid(acc, hp, dinv, b, wn):
    def body(acc_ref, hp_ref, dinv_ref, b_ref, w_ref, out_ref):
        a = acc_ref[...]
        dinv = dinv_ref[...]
        s = a[0] + a[1] + hp_ref[...]
        h = jnp.maximum(dinv * s + b_ref[...], 0.0)
        out_ref[...] = dinv * _dot(h, w_ref[...])

    return pl.pallas_call(
        body,
        grid=(N // BR,),
        in_specs=[
            pl.BlockSpec((NC, BR, DW), lambda i: (0, i, 0)),
            pl.BlockSpec((BR, DW), lambda i: (i, 0)),
            pl.BlockSpec((BR, 1), lambda i: (i, 0)),
            pl.BlockSpec((1, DW), lambda i: (0, 0)),
            pl.BlockSpec((DW, DW), lambda i: (0, 0)),
        ],
        out_specs=pl.BlockSpec((BR, DW), lambda i: (i, 0)),
        out_shape=jax.ShapeDtypeStruct((N, DW), jnp.float32),
    )(acc, hp, dinv, b, wn)


def _tc_stage3(acc, hp, dinv, b):
    def body(acc_ref, hp_ref, dinv_ref, b_ref, out_ref):
        a = acc_ref[...]
        full = dinv_ref[...] * (a[0] + a[1] + hp_ref[...])
        logits = full[:, 0:K] + b_ref[...]
        m = jnp.max(logits, axis=1, keepdims=True)
        e = jnp.exp(logits - m)
        out_ref[...] = e / jnp.sum(e, axis=1, keepdims=True)

    return pl.pallas_call(
        body,
        grid=(N // BR,),
        in_specs=[
            pl.BlockSpec((NC, BR, DW), lambda i: (0, i, 0)),
            pl.BlockSpec((BR, DW), lambda i: (i, 0)),
            pl.BlockSpec((BR, 1), lambda i: (i, 0)),
            pl.BlockSpec((1, K), lambda i: (0, 0)),
        ],
        out_specs=pl.BlockSpec((BR, K), lambda i: (i, 0)),
        out_shape=jax.ShapeDtypeStruct((N, K), jnp.float32),
    )(acc, hp, dinv, b)


def kernel(x, edge_index, W1, b1, W2, b2, W3, b3):
    src = edge_index[0].reshape(NW, EPT_RAW)
    dst = edge_index[1].reshape(NW, EPT_RAW)
    pad = EPT - EPT_RAW
    src = jnp.pad(src, ((0, 0), (0, pad))).reshape(NW, NSEG, NCHS, CHUNK)
    dst = jnp.pad(dst, ((0, 0), (0, pad)),
                  constant_values=TRASH).reshape(NW, NSEG, NCHS, CHUNK)

    zdw = jnp.zeros((CHUNK, DW), jnp.float32)
    onesdw = jnp.ones((CHUNK, DW), jnp.float32)

    w2p = jnp.zeros((DW, DW), jnp.float32).at[:H1, :H2].set(W2)
    w3p = jnp.zeros((DW, DW), jnp.float32).at[:H2, :K].set(W3)
    b2p = jnp.zeros((1, DW), jnp.float32).at[0, :H2].set(b2)

    agg = _sc_aggregate()

    dp = _sc_degree()(dst.reshape(NW, NCH, CHUNK), zdw, onesdw)

    hp1, dinv = _tc_stage1(x, W1, dp)
    acc1 = agg(hp1, src, dst, zdw)
    hp2 = _tc_stage_mid(acc1, hp1, dinv, b1.reshape(1, H1), w2p)
    acc2 = agg(hp2, src, dst, zdw)
    hp3 = _tc_stage_mid(acc2, hp2, dinv, b2p, w3p)
    acc3 = agg(hp3, src, dst, zdw)

    return _tc_stage3(acc3, hp3, dinv, b3.reshape(1, K))
